# baseline (speedup 1.0000x reference)
import functools

import jax
import jax.numpy as jnp
from jax import lax
from jax.experimental import pallas as pl
from jax.experimental.pallas import tpu as pltpu
from jax.experimental.pallas import tpu_sc as plsc

N_BIO = 1000000
N_FACTORS = 64
BATCH = 16384

_info = plsc.get_sparse_core_info()
_NC = _info.num_cores
_NS = _info.num_subcores
_NW = _NC * _NS
_BPW = BATCH // _NW
_CH = 128
_NCH = _BPW // _CH

_mesh = plsc.VectorSubcoreMesh(core_axis_name="c", subcore_axis_name="s")


@functools.partial(
    pl.kernel,
    mesh=_mesh,
    out_type=jax.ShapeDtypeStruct((_NW, _NCH, _CH, N_FACTORS), jnp.float32),
    scratch_types=[
        pltpu.VMEM((_NCH, _CH), jnp.int32),
        pltpu.VMEM((_NCH, _CH, N_FACTORS), jnp.float32),
        pltpu.SemaphoreType.DMA,
    ],
    compiler_params=pltpu.CompilerParams(use_tc_tiling_on_sc=False),
)
def _gather_kernel(idx_hbm, table_hbm, out_hbm, idx_v, rows_v, sem):
    wid = lax.axis_index("s") * _NC + lax.axis_index("c")
    pltpu.sync_copy(idx_hbm.at[wid], idx_v)
    copies = [
        pltpu.async_copy(table_hbm.at[idx_v.at[j]], rows_v.at[j], sem)
        for j in range(_NCH)
    ]
    for c in copies:
        c.wait()
    pltpu.sync_copy(rows_v, out_hbm.at[wid])


def kernel(idxs, bio_factors):
    idx3 = idxs.astype(jnp.int32).reshape(_NW, _NCH, _CH)
    out = _gather_kernel(idx3, bio_factors)
    return out.reshape(BATCH, N_FACTORS)
